# Initial kernel scaffold; baseline (speedup 1.0000x reference)
import jax
import jax.numpy as jnp
from jax.experimental import pallas as pl


def kernel(pos, batch):
    raise NotImplementedError("write your pallas kernel here")


# speedup vs baseline: 98.5768x; 98.5768x over previous
import functools

import jax
import jax.numpy as jnp
from jax import lax
from jax.experimental import pallas as pl
from jax.experimental.pallas import tpu as pltpu
from jax.experimental.pallas import tpu_sc as plsc

N = 4096
K = 32
R2 = 25.0
NB = 32
L = 16
NC, NS = 2, 16
NW = NC * NS
RPW = N // NW
NCHUNK = N // L
INF = float("inf")


def _lexless(ka, va, kb, vb):
    return (ka < kb) | ((ka == kb) & (va < vb))


def _merge16into32(T0k, T0v, T1k, T1v, Ck, Cv):
    rCk = jnp.flip(Ck, 0)
    rCv = jnp.flip(Cv, 0)
    lt = _lexless(T1k, T1v, rCk, rCv)
    L1k = jnp.where(lt, T1k, rCk)
    L1v = jnp.where(lt, T1v, rCv)
    lt2 = _lexless(T0k, T0v, L1k, L1v)
    P0k = jnp.where(lt2, T0k, L1k)
    P0v = jnp.where(lt2, T0v, L1v)
    P1k = jnp.where(lt2, L1k, T0k)
    P1v = jnp.where(lt2, L1v, T0v)
    T0k, T0v = plsc.sort_key_val(P0k, P0v)
    T1k, T1v = plsc.sort_key_val(P1k, P1v)
    return T0k, T0v, T1k, T1v


def _splat_lane(v, lane, fill):
    iota = lax.iota(jnp.int32, L)
    s = jnp.max(jnp.where(iota == lane, v, fill))
    return jnp.broadcast_to(s, (L,))


def _sqrt16(x):
    i = plsc.bitcast(x, jnp.int32)
    i = jnp.int32(0x5F3759DF) - (i >> 1)
    y = plsc.bitcast(i, jnp.float32)
    half_x = jnp.float32(0.5) * x
    for _ in range(3):
        y = y * (jnp.float32(1.5) - half_x * y * y)
    return x * y


def _tec_body(x_hbm, y_hbm, z_hbm, b_hbm, src_hbm, tgt_hbm, w_hbm,
              xv, yv, zv, bv, sqv, lo_t, hi_t, src_v, tgt_v, w_v):
    wid = lax.axis_index("s") * NC + lax.axis_index("c")
    r0 = wid * RPW
    iota = lax.iota(jnp.int32, L)

    pltpu.sync_copy(x_hbm, xv)
    pltpu.sync_copy(y_hbm, yv)
    pltpu.sync_copy(z_hbm, zv)
    bv[pl.ds(0, L)] = jnp.full((L,), -1, jnp.int32)
    bv[pl.ds(L + N, L)] = jnp.full((L,), NB, jnp.int32)
    pltpu.sync_copy(b_hbm, bv.at[pl.ds(L, N)])

    def sq_body(i, carry):
        off = i * L
        x = xv[pl.ds(off, L)]
        y = yv[pl.ds(off, L)]
        z = zv[pl.ds(off, L)]
        sqv[pl.ds(off, L)] = x * x + y * y + z * z
        return carry

    lax.fori_loop(0, NCHUNK, sq_body, 0)

    def bnd_body(c, carry):
        off = c * L
        cur = bv[pl.ds(L + off, L)]
        prv = bv[pl.ds(L + off - 1, L)]
        nxt = bv[pl.ds(L + off + 1, L)]
        gidx = iota + off
        plsc.store_scatter(lo_t, [cur], gidx, mask=cur != prv)
        plsc.store_scatter(hi_t, [cur], gidx + 1, mask=cur != nxt)
        return carry

    lax.fori_loop(0, NCHUNK, bnd_body, 0)

    z16 = jnp.zeros((L,), jnp.int32)
    inf16 = jnp.full((L,), INF, jnp.float32)

    def row_body(rr, carry):
        gi = r0 + rr
        gic = (gi // L) * L
        lane = gi - gic
        xi = _splat_lane(xv[pl.ds(gic, L)], lane, -INF)
        yi = _splat_lane(yv[pl.ds(gic, L)], lane, -INF)
        zi = _splat_lane(zv[pl.ds(gic, L)], lane, -INF)
        sqi = _splat_lane(sqv[pl.ds(gic, L)], lane, -INF)
        bsp = _splat_lane(bv[pl.ds(L + gic, L)], lane, jnp.int32(-1))
        gisp = jnp.broadcast_to(gi, (L,))

        lo = jnp.max(plsc.load_gather(lo_t, [bsp]))
        hi = jnp.max(plsc.load_gather(hi_t, [bsp]))
        c0 = lo // L
        c1 = (hi + (L - 1)) // L

        def chunk_body(c, T):
            T0k, T0v, T1k, T1v = T
            off = c * L
            bx = xv[pl.ds(off, L)]
            by = yv[pl.ds(off, L)]
            bz = zv[pl.ds(off, L)]
            bb = bv[pl.ds(L + off, L)]
            bsq = sqv[pl.ds(off, L)]
            dot = xi * bx + yi * by + zi * bz
            d2 = jnp.maximum((sqi + bsq) - 2.0 * dot, 0.0)
            vidx = iota + off
            m = (bb == bsp) & (vidx != gisp) & (d2 <= R2)
            key = jnp.where(m, d2, INF)
            Ck, Cv = plsc.sort_key_val(key, vidx)
            return _merge16into32(T0k, T0v, T1k, T1v, Ck, Cv)

        T0k, T0v, T1k, T1v = lax.fori_loop(
            c0, c1, chunk_body, (inf16, z16, inf16, z16))

        base = rr * K
        for half, (tk, tv) in enumerate(((T0k, T0v), (T1k, T1v))):
            keep = tk <= R2
            good = keep & (tk > 0.0)
            safe = jnp.where(good, tk, jnp.float32(1.0))
            wgt = jnp.where(good, _sqrt16(safe), jnp.float32(0.0))
            off = base + half * L
            src_v[pl.ds(off, L)] = jnp.where(keep, tv, -1)
            tgt_v[pl.ds(off, L)] = jnp.where(keep, gisp, -1)
            w_v[pl.ds(off, L)] = wgt
        return carry

    lax.fori_loop(0, RPW, row_body, 0)

    out0 = r0 * K
    pltpu.sync_copy(src_v, src_hbm.at[pl.ds(out0, RPW * K)])
    pltpu.sync_copy(tgt_v, tgt_hbm.at[pl.ds(out0, RPW * K)])
    pltpu.sync_copy(w_v, w_hbm.at[pl.ds(out0, RPW * K)])


@jax.jit
def kernel(pos, batch):
    x = pos[:, 0]
    y = pos[:, 1]
    z = pos[:, 2]
    b = batch.astype(jnp.int32)

    mesh = plsc.VectorSubcoreMesh(core_axis_name="c", subcore_axis_name="s")
    run = functools.partial(
        pl.kernel,
        out_type=[
            jax.ShapeDtypeStruct((N * K,), jnp.int32),
            jax.ShapeDtypeStruct((N * K,), jnp.int32),
            jax.ShapeDtypeStruct((N * K,), jnp.float32),
        ],
        mesh=mesh,
        compiler_params=pltpu.CompilerParams(needs_layout_passes=False),
        scratch_types=[
            pltpu.VMEM((N,), jnp.float32),
            pltpu.VMEM((N,), jnp.float32),
            pltpu.VMEM((N,), jnp.float32),
            pltpu.VMEM((N + 2 * L,), jnp.int32),
            pltpu.VMEM((N,), jnp.float32),
            pltpu.VMEM((NB,), jnp.int32),
            pltpu.VMEM((NB,), jnp.int32),
            pltpu.VMEM((RPW * K,), jnp.int32),
            pltpu.VMEM((RPW * K,), jnp.int32),
            pltpu.VMEM((RPW * K,), jnp.float32),
        ],
    )(_tec_body)
    src, tgt, w = run(x, y, z, b)
    edge_index = jnp.stack([src, tgt])
    return edge_index, w


# speedup vs baseline: 100.1933x; 1.0164x over previous
import functools

import jax
import jax.numpy as jnp
from jax import lax
from jax.experimental import pallas as pl
from jax.experimental.pallas import tpu as pltpu
from jax.experimental.pallas import tpu_sc as plsc

N = 4096
K = 32
R2 = 25.0
NB = 32
L = 16
NC, NS = 2, 16
NW = NC * NS
RPW = N // NW
NCHUNK = N // L
INF = float("inf")


def _lexless(ka, va, kb, vb):
    return (ka < kb) | ((ka == kb) & (va < vb))


def _merge16to32(Ck, Cv, Dk, Dv):
    rDk = jnp.flip(Dk, 0)
    rDv = jnp.flip(Dv, 0)
    lt = _lexless(Ck, Cv, rDk, rDv)
    P0k = jnp.where(lt, Ck, rDk)
    P0v = jnp.where(lt, Cv, rDv)
    P1k = jnp.where(lt, rDk, Ck)
    P1v = jnp.where(lt, rDv, Cv)
    S0k, S0v = plsc.sort_key_val(P0k, P0v)
    S1k, S1v = plsc.sort_key_val(P1k, P1v)
    return S0k, S0v, S1k, S1v


def _merge32keep32(T0k, T0v, T1k, T1v, S0k, S0v, S1k, S1v):
    rS0k = jnp.flip(S0k, 0)
    rS0v = jnp.flip(S0v, 0)
    rS1k = jnp.flip(S1k, 0)
    rS1v = jnp.flip(S1v, 0)
    ltA = _lexless(T0k, T0v, rS1k, rS1v)
    L0k = jnp.where(ltA, T0k, rS1k)
    L0v = jnp.where(ltA, T0v, rS1v)
    ltB = _lexless(T1k, T1v, rS0k, rS0v)
    L1k = jnp.where(ltB, T1k, rS0k)
    L1v = jnp.where(ltB, T1v, rS0v)
    lt2 = _lexless(L0k, L0v, L1k, L1v)
    P0k = jnp.where(lt2, L0k, L1k)
    P0v = jnp.where(lt2, L0v, L1v)
    P1k = jnp.where(lt2, L1k, L0k)
    P1v = jnp.where(lt2, L1v, L0v)
    T0k, T0v = plsc.sort_key_val(P0k, P0v)
    T1k, T1v = plsc.sort_key_val(P1k, P1v)
    return T0k, T0v, T1k, T1v


def _sqrt16(x):
    i = plsc.bitcast(x, jnp.int32)
    i = jnp.int32(0x5F3759DF) - (i >> 1)
    y = plsc.bitcast(i, jnp.float32)
    half_x = jnp.float32(0.5) * x
    for _ in range(3):
        y = y * (jnp.float32(1.5) - half_x * y * y)
    return x * y


def _tec_body(x_hbm, y_hbm, z_hbm, b_hbm, src_hbm, tgt_hbm, w_hbm,
              xv, yv, zv, bv, sqv, lo_t, hi_t, src_v, tgt_v, w_v):
    wid = lax.axis_index("s") * NC + lax.axis_index("c")
    r0 = wid * RPW
    iota = lax.iota(jnp.int32, L)

    pltpu.sync_copy(x_hbm, xv)
    pltpu.sync_copy(y_hbm, yv)
    pltpu.sync_copy(z_hbm, zv)
    bv[pl.ds(0, L)] = jnp.full((L,), -1, jnp.int32)
    bv[pl.ds(L + N, L)] = jnp.full((L,), NB, jnp.int32)
    pltpu.sync_copy(b_hbm, bv.at[pl.ds(L, N)])

    def sq_body(i, carry):
        off = i * L
        x = xv[pl.ds(off, L)]
        y = yv[pl.ds(off, L)]
        z = zv[pl.ds(off, L)]
        sqv[pl.ds(off, L)] = x * x + y * y + z * z
        return carry

    lax.fori_loop(0, NCHUNK, sq_body, 0)

    def bnd_body(c, carry):
        off = c * L
        cur = bv[pl.ds(L + off, L)]
        prv = bv[pl.ds(L + off - 1, L)]
        nxt = bv[pl.ds(L + off + 1, L)]
        gidx = iota + off
        plsc.store_scatter(lo_t, [cur], gidx, mask=cur != prv)
        plsc.store_scatter(hi_t, [cur], gidx + 1, mask=cur != nxt)
        return carry

    lax.fori_loop(0, NCHUNK, bnd_body, 0)

    z16 = jnp.zeros((L,), jnp.int32)
    inf16 = jnp.full((L,), INF, jnp.float32)

    def row_body(rr, carry):
        gi = r0 + rr
        gisp = jnp.broadcast_to(gi, (L,))
        xi = plsc.load_gather(xv, [gisp])
        yi = plsc.load_gather(yv, [gisp])
        zi = plsc.load_gather(zv, [gisp])
        sqi = plsc.load_gather(sqv, [gisp])
        bsp = plsc.load_gather(bv, [gisp + L])

        lo = jnp.max(plsc.load_gather(lo_t, [bsp]))
        hi = jnp.max(plsc.load_gather(hi_t, [bsp]))
        c0 = lo // L
        c1 = (hi + (L - 1)) // L

        def chunk16(c, cvalid):
            off = c * L
            bx = xv[pl.ds(off, L)]
            by = yv[pl.ds(off, L)]
            bz = zv[pl.ds(off, L)]
            bb = bv[pl.ds(L + off, L)]
            bsq = sqv[pl.ds(off, L)]
            dot = xi * bx + yi * by + zi * bz
            d2 = jnp.maximum((sqi + bsq) - 2.0 * dot, 0.0)
            vidx = iota + off
            m = (bb == bsp) & (vidx != gisp) & (d2 <= R2) & cvalid
            key = jnp.where(m, d2, INF)
            return plsc.sort_key_val(key, vidx)

        def pair_body(p, T):
            ca = c0 + 2 * p
            cb = ca + 1
            Ck, Cv = chunk16(ca, True)
            Dk, Dv = chunk16(jnp.minimum(cb, NCHUNK - 1), cb < c1)
            S = _merge16to32(Ck, Cv, Dk, Dv)
            return _merge32keep32(*T, *S)

        npairs = (c1 - c0 + 1) // 2
        T0k, T0v, T1k, T1v = lax.fori_loop(
            0, npairs, pair_body, (inf16, z16, inf16, z16))

        base = rr * K
        for half, (tk, tv) in enumerate(((T0k, T0v), (T1k, T1v))):
            keep = tk <= R2
            good = keep & (tk > 0.0)
            safe = jnp.where(good, tk, jnp.float32(1.0))
            wgt = jnp.where(good, _sqrt16(safe), jnp.float32(0.0))
            off = base + half * L
            src_v[pl.ds(off, L)] = jnp.where(keep, tv, -1)
            tgt_v[pl.ds(off, L)] = jnp.where(keep, gisp, -1)
            w_v[pl.ds(off, L)] = wgt
        return carry

    lax.fori_loop(0, RPW, row_body, 0)

    out0 = r0 * K
    pltpu.sync_copy(src_v, src_hbm.at[pl.ds(out0, RPW * K)])
    pltpu.sync_copy(tgt_v, tgt_hbm.at[pl.ds(out0, RPW * K)])
    pltpu.sync_copy(w_v, w_hbm.at[pl.ds(out0, RPW * K)])


@jax.jit
def kernel(pos, batch):
    x = pos[:, 0]
    y = pos[:, 1]
    z = pos[:, 2]
    b = batch.astype(jnp.int32)

    mesh = plsc.VectorSubcoreMesh(core_axis_name="c", subcore_axis_name="s")
    run = functools.partial(
        pl.kernel,
        out_type=[
            jax.ShapeDtypeStruct((N * K,), jnp.int32),
            jax.ShapeDtypeStruct((N * K,), jnp.int32),
            jax.ShapeDtypeStruct((N * K,), jnp.float32),
        ],
        mesh=mesh,
        compiler_params=pltpu.CompilerParams(needs_layout_passes=False),
        scratch_types=[
            pltpu.VMEM((N,), jnp.float32),
            pltpu.VMEM((N,), jnp.float32),
            pltpu.VMEM((N,), jnp.float32),
            pltpu.VMEM((N + 2 * L,), jnp.int32),
            pltpu.VMEM((N,), jnp.float32),
            pltpu.VMEM((NB,), jnp.int32),
            pltpu.VMEM((NB,), jnp.int32),
            pltpu.VMEM((RPW * K,), jnp.int32),
            pltpu.VMEM((RPW * K,), jnp.int32),
            pltpu.VMEM((RPW * K,), jnp.float32),
        ],
    )(_tec_body)
    src, tgt, w = run(x, y, z, b)
    edge_index = jnp.stack([src, tgt])
    return edge_index, w


# speedup vs baseline: 101.0679x; 1.0087x over previous
import functools

import jax
import jax.numpy as jnp
from jax import lax
from jax.experimental import pallas as pl
from jax.experimental.pallas import tpu as pltpu
from jax.experimental.pallas import tpu_sc as plsc

N = 4096
K = 32
R2 = 25.0
NB = 32
L = 16
NC, NS = 2, 16
NW = NC * NS
RPW = N // NW
NCHUNK = N // L
INF = float("inf")


def _lexless(ka, va, kb, vb):
    return (ka < kb) | ((ka == kb) & (va < vb))


def _merge16to32(Ck, Cv, Dk, Dv):
    rDk = jnp.flip(Dk, 0)
    rDv = jnp.flip(Dv, 0)
    lt = _lexless(Ck, Cv, rDk, rDv)
    P0k = jnp.where(lt, Ck, rDk)
    P0v = jnp.where(lt, Cv, rDv)
    P1k = jnp.where(lt, rDk, Ck)
    P1v = jnp.where(lt, rDv, Cv)
    S0k, S0v = plsc.sort_key_val(P0k, P0v)
    S1k, S1v = plsc.sort_key_val(P1k, P1v)
    return S0k, S0v, S1k, S1v


def _merge32keep32(T0k, T0v, T1k, T1v, S0k, S0v, S1k, S1v):
    rS0k = jnp.flip(S0k, 0)
    rS0v = jnp.flip(S0v, 0)
    rS1k = jnp.flip(S1k, 0)
    rS1v = jnp.flip(S1v, 0)
    ltA = _lexless(T0k, T0v, rS1k, rS1v)
    L0k = jnp.where(ltA, T0k, rS1k)
    L0v = jnp.where(ltA, T0v, rS1v)
    ltB = _lexless(T1k, T1v, rS0k, rS0v)
    L1k = jnp.where(ltB, T1k, rS0k)
    L1v = jnp.where(ltB, T1v, rS0v)
    lt2 = _lexless(L0k, L0v, L1k, L1v)
    P0k = jnp.where(lt2, L0k, L1k)
    P0v = jnp.where(lt2, L0v, L1v)
    P1k = jnp.where(lt2, L1k, L0k)
    P1v = jnp.where(lt2, L1v, L0v)
    T0k, T0v = plsc.sort_key_val(P0k, P0v)
    T1k, T1v = plsc.sort_key_val(P1k, P1v)
    return T0k, T0v, T1k, T1v


def _sqrt16(x):
    i = plsc.bitcast(x, jnp.int32)
    i = jnp.int32(0x5F3759DF) - (i >> 1)
    y = plsc.bitcast(i, jnp.float32)
    half_x = jnp.float32(0.5) * x
    for _ in range(3):
        y = y * (jnp.float32(1.5) - half_x * y * y)
    return x * y


def _tec_body(x_hbm, y_hbm, z_hbm, b_hbm, src_hbm, tgt_hbm, w_hbm,
              xv, yv, zv, bv, sqv, lo_t, hi_t, src_v, tgt_v, w_v):
    wid = lax.axis_index("s") * NC + lax.axis_index("c")
    r0 = wid * RPW
    iota = lax.iota(jnp.int32, L)

    pltpu.sync_copy(x_hbm, xv)
    pltpu.sync_copy(y_hbm, yv)
    pltpu.sync_copy(z_hbm, zv)
    bv[pl.ds(0, L)] = jnp.full((L,), -1, jnp.int32)
    bv[pl.ds(L + N, L)] = jnp.full((L,), NB, jnp.int32)
    pltpu.sync_copy(b_hbm, bv.at[pl.ds(L, N)])

    def sq_body(i, carry):
        off = i * L
        x = xv[pl.ds(off, L)]
        y = yv[pl.ds(off, L)]
        z = zv[pl.ds(off, L)]
        sqv[pl.ds(off, L)] = x * x + y * y + z * z
        return carry

    lax.fori_loop(0, NCHUNK, sq_body, 0)

    def bnd_body(c, carry):
        off = c * L
        cur = bv[pl.ds(L + off, L)]
        prv = bv[pl.ds(L + off - 1, L)]
        nxt = bv[pl.ds(L + off + 1, L)]
        gidx = iota + off
        plsc.store_scatter(lo_t, [cur], gidx, mask=cur != prv)
        plsc.store_scatter(hi_t, [cur], gidx + 1, mask=cur != nxt)
        return carry

    lax.fori_loop(0, NCHUNK, bnd_body, 0)

    z16 = jnp.zeros((L,), jnp.int32)
    inf16 = jnp.full((L,), INF, jnp.float32)

    def row_body(rr):
        gi = r0 + rr
        gisp = jnp.broadcast_to(gi, (L,))
        xi = plsc.load_gather(xv, [gisp])
        yi = plsc.load_gather(yv, [gisp])
        zi = plsc.load_gather(zv, [gisp])
        sqi = plsc.load_gather(sqv, [gisp])
        bsp = plsc.load_gather(bv, [gisp + L])

        lo = jnp.max(plsc.load_gather(lo_t, [bsp]))
        hi = jnp.max(plsc.load_gather(hi_t, [bsp]))
        c0 = lo // L
        c1 = (hi + (L - 1)) // L

        def chunk16(c, cvalid):
            off = c * L
            bx = xv[pl.ds(off, L)]
            by = yv[pl.ds(off, L)]
            bz = zv[pl.ds(off, L)]
            bb = bv[pl.ds(L + off, L)]
            bsq = sqv[pl.ds(off, L)]
            dot = xi * bx + yi * by + zi * bz
            d2 = jnp.maximum((sqi + bsq) - 2.0 * dot, 0.0)
            vidx = iota + off
            m = (bb == bsp) & (vidx != gisp) & (d2 <= R2) & cvalid
            key = jnp.where(m, d2, INF)
            return plsc.sort_key_val(key, vidx)

        def pair_body(p, T):
            ca = c0 + 2 * p
            cb = ca + 1
            Ck, Cv = chunk16(ca, True)
            Dk, Dv = chunk16(jnp.minimum(cb, NCHUNK - 1), cb < c1)
            S = _merge16to32(Ck, Cv, Dk, Dv)
            return _merge32keep32(*T, *S)

        npairs = (c1 - c0 + 1) // 2
        T0k, T0v, T1k, T1v = lax.fori_loop(
            0, npairs, pair_body, (inf16, z16, inf16, z16))

        base = rr * K
        for half, (tk, tv) in enumerate(((T0k, T0v), (T1k, T1v))):
            keep = tk <= R2
            good = keep & (tk > 0.0)
            safe = jnp.where(good, tk, jnp.float32(1.0))
            wgt = jnp.where(good, _sqrt16(safe), jnp.float32(0.0))
            off = base + half * L
            src_v[pl.ds(off, L)] = jnp.where(keep, tv, -1)
            tgt_v[pl.ds(off, L)] = jnp.where(keep, gisp, -1)
            w_v[pl.ds(off, L)] = wgt

    plsc.parallel_loop(0, RPW, 1, unroll=2)(row_body)

    out0 = r0 * K
    pltpu.sync_copy(src_v, src_hbm.at[pl.ds(out0, RPW * K)])
    pltpu.sync_copy(tgt_v, tgt_hbm.at[pl.ds(out0, RPW * K)])
    pltpu.sync_copy(w_v, w_hbm.at[pl.ds(out0, RPW * K)])


@jax.jit
def kernel(pos, batch):
    x = pos[:, 0]
    y = pos[:, 1]
    z = pos[:, 2]
    b = batch.astype(jnp.int32)

    mesh = plsc.VectorSubcoreMesh(core_axis_name="c", subcore_axis_name="s")
    run = functools.partial(
        pl.kernel,
        out_type=[
            jax.ShapeDtypeStruct((N * K,), jnp.int32),
            jax.ShapeDtypeStruct((N * K,), jnp.int32),
            jax.ShapeDtypeStruct((N * K,), jnp.float32),
        ],
        mesh=mesh,
        compiler_params=pltpu.CompilerParams(needs_layout_passes=False),
        scratch_types=[
            pltpu.VMEM((N,), jnp.float32),
            pltpu.VMEM((N,), jnp.float32),
            pltpu.VMEM((N,), jnp.float32),
            pltpu.VMEM((N + 2 * L,), jnp.int32),
            pltpu.VMEM((N,), jnp.float32),
            pltpu.VMEM((NB,), jnp.int32),
            pltpu.VMEM((NB,), jnp.int32),
            pltpu.VMEM((RPW * K,), jnp.int32),
            pltpu.VMEM((RPW * K,), jnp.int32),
            pltpu.VMEM((RPW * K,), jnp.float32),
        ],
    )(_tec_body)
    src, tgt, w = run(x, y, z, b)
    edge_index = jnp.stack([src, tgt])
    return edge_index, w


# speedup vs baseline: 126.5166x; 1.2518x over previous
import functools

import jax
import jax.numpy as jnp
from jax import lax
from jax.experimental import pallas as pl
from jax.experimental.pallas import tpu as pltpu
from jax.experimental.pallas import tpu_sc as plsc

N = 4096
K = 32
R2 = 25.0
NB = 32
L = 16
NC, NS = 2, 16
NW = NC * NS
RPW = N // NW
NCHUNK = N // L
INF = float("inf")


def _lexless(ka, va, kb, vb):
    return (ka < kb) | ((ka == kb) & (va < vb))


def _merge16to32(Ck, Cv, Dk, Dv):
    rDk = jnp.flip(Dk, 0)
    rDv = jnp.flip(Dv, 0)
    lt = _lexless(Ck, Cv, rDk, rDv)
    P0k = jnp.where(lt, Ck, rDk)
    P0v = jnp.where(lt, Cv, rDv)
    P1k = jnp.where(lt, rDk, Ck)
    P1v = jnp.where(lt, rDv, Cv)
    S0k, S0v = plsc.sort_key_val(P0k, P0v)
    S1k, S1v = plsc.sort_key_val(P1k, P1v)
    return S0k, S0v, S1k, S1v


def _merge32keep32(T0k, T0v, T1k, T1v, S0k, S0v, S1k, S1v):
    rS0k = jnp.flip(S0k, 0)
    rS0v = jnp.flip(S0v, 0)
    rS1k = jnp.flip(S1k, 0)
    rS1v = jnp.flip(S1v, 0)
    ltA = _lexless(T0k, T0v, rS1k, rS1v)
    L0k = jnp.where(ltA, T0k, rS1k)
    L0v = jnp.where(ltA, T0v, rS1v)
    ltB = _lexless(T1k, T1v, rS0k, rS0v)
    L1k = jnp.where(ltB, T1k, rS0k)
    L1v = jnp.where(ltB, T1v, rS0v)
    lt2 = _lexless(L0k, L0v, L1k, L1v)
    P0k = jnp.where(lt2, L0k, L1k)
    P0v = jnp.where(lt2, L0v, L1v)
    P1k = jnp.where(lt2, L1k, L0k)
    P1v = jnp.where(lt2, L1v, L0v)
    T0k, T0v = plsc.sort_key_val(P0k, P0v)
    T1k, T1v = plsc.sort_key_val(P1k, P1v)
    return T0k, T0v, T1k, T1v


def _sqrt16(x):
    i = plsc.bitcast(x, jnp.int32)
    i = jnp.int32(0x5F3759DF) - (i >> 1)
    y = plsc.bitcast(i, jnp.float32)
    half_x = jnp.float32(0.5) * x
    for _ in range(3):
        y = y * (jnp.float32(1.5) - half_x * y * y)
    return x * y


def _tec_body(x_hbm, y_hbm, z_hbm, b_hbm, src_hbm, tgt_hbm, w_hbm,
              xv, yv, zv, bv, sqv, lo_t, hi_t, src_v, tgt_v, w_v):
    wid = lax.axis_index("s") * NC + lax.axis_index("c")
    r0 = wid * RPW
    iota = lax.iota(jnp.int32, L)

    pltpu.sync_copy(x_hbm, xv)
    pltpu.sync_copy(y_hbm, yv)
    pltpu.sync_copy(z_hbm, zv)
    bv[pl.ds(0, L)] = jnp.full((L,), -1, jnp.int32)
    bv[pl.ds(L + N, L)] = jnp.full((L,), NB, jnp.int32)
    pltpu.sync_copy(b_hbm, bv.at[pl.ds(L, N)])

    def sq_body(i, carry):
        off = i * L
        x = xv[pl.ds(off, L)]
        y = yv[pl.ds(off, L)]
        z = zv[pl.ds(off, L)]
        sqv[pl.ds(off, L)] = x * x + y * y + z * z
        return carry

    lax.fori_loop(0, NCHUNK, sq_body, 0)

    def bnd_body(c, carry):
        off = c * L
        cur = bv[pl.ds(L + off, L)]
        prv = bv[pl.ds(L + off - 1, L)]
        nxt = bv[pl.ds(L + off + 1, L)]
        gidx = iota + off
        plsc.store_scatter(lo_t, [cur], gidx, mask=cur != prv)
        plsc.store_scatter(hi_t, [cur], gidx + 1, mask=cur != nxt)
        return carry

    lax.fori_loop(0, NCHUNK, bnd_body, 0)

    z16 = jnp.zeros((L,), jnp.int32)
    inf16 = jnp.full((L,), INF, jnp.float32)

    def rowpair_body(rp):
        ga = r0 + 2 * rp
        gb = ga + 1

        def row_ctx(gi):
            gisp = jnp.broadcast_to(gi, (L,))
            xi = plsc.load_gather(xv, [gisp])
            yi = plsc.load_gather(yv, [gisp])
            zi = plsc.load_gather(zv, [gisp])
            sqi = plsc.load_gather(sqv, [gisp])
            bsp = plsc.load_gather(bv, [gisp + L])
            lo = jnp.max(plsc.load_gather(lo_t, [bsp]))
            hi = jnp.max(plsc.load_gather(hi_t, [bsp]))
            return gisp, xi, yi, zi, sqi, bsp, lo, hi

        ctx_a = row_ctx(ga)
        ctx_b = row_ctx(gb)
        c0 = jnp.minimum(ctx_a[6], ctx_b[6]) // L
        c1 = (jnp.maximum(ctx_a[7], ctx_b[7]) + (L - 1)) // L

        def chunk16(ctx, off, vidx, bx, by, bz, bb, bsq, cvalid):
            gisp, xi, yi, zi, sqi, bsp = ctx[:6]
            dot = xi * bx + yi * by + zi * bz
            d2 = jnp.maximum((sqi + bsq) - 2.0 * dot, 0.0)
            m = (bb == bsp) & (vidx != gisp) & (d2 <= R2) & cvalid
            key = jnp.where(m, d2, INF)
            return plsc.sort_key_val(key, vidx)

        def pair_body(p, T):
            Ta, Tb = T[:4], T[4:]
            ca = c0 + 2 * p
            cb = jnp.minimum(ca + 1, NCHUNK - 1)
            bvalid = ca + 1 < c1
            offa = ca * L
            offb = cb * L
            la = (xv[pl.ds(offa, L)], yv[pl.ds(offa, L)], zv[pl.ds(offa, L)],
                  bv[pl.ds(L + offa, L)], sqv[pl.ds(offa, L)])
            lb = (xv[pl.ds(offb, L)], yv[pl.ds(offb, L)], zv[pl.ds(offb, L)],
                  bv[pl.ds(L + offb, L)], sqv[pl.ds(offb, L)])
            via = iota + offa
            vib = iota + offb
            CkA, CvA = chunk16(ctx_a, offa, via, *la, True)
            DkA, DvA = chunk16(ctx_a, offb, vib, *lb, bvalid)
            CkB, CvB = chunk16(ctx_b, offa, via, *la, True)
            DkB, DvB = chunk16(ctx_b, offb, vib, *lb, bvalid)
            SA = _merge16to32(CkA, CvA, DkA, DvA)
            SB = _merge16to32(CkB, CvB, DkB, DvB)
            Ta = _merge32keep32(*Ta, *SA)
            Tb = _merge32keep32(*Tb, *SB)
            return Ta + Tb

        npairs = (c1 - c0 + 1) // 2
        init = (inf16, z16, inf16, z16)
        T = lax.fori_loop(0, npairs, pair_body, init + init)

        for gisp_r, Tr, rr in ((ctx_a[0], T[:4], 2 * rp), (ctx_b[0], T[4:], 2 * rp + 1)):
            T0k, T0v, T1k, T1v = Tr
            base = rr * K
            for half, (tk, tv) in enumerate(((T0k, T0v), (T1k, T1v))):
                keep = tk <= R2
                good = keep & (tk > 0.0)
                safe = jnp.where(good, tk, jnp.float32(1.0))
                wgt = jnp.where(good, _sqrt16(safe), jnp.float32(0.0))
                off = base + half * L
                src_v[pl.ds(off, L)] = jnp.where(keep, tv, -1)
                tgt_v[pl.ds(off, L)] = jnp.where(keep, gisp_r, -1)
                w_v[pl.ds(off, L)] = wgt

    plsc.parallel_loop(0, RPW // 2, 1, unroll=2)(rowpair_body)

    out0 = r0 * K
    pltpu.sync_copy(src_v, src_hbm.at[pl.ds(out0, RPW * K)])
    pltpu.sync_copy(tgt_v, tgt_hbm.at[pl.ds(out0, RPW * K)])
    pltpu.sync_copy(w_v, w_hbm.at[pl.ds(out0, RPW * K)])


@jax.jit
def kernel(pos, batch):
    x = pos[:, 0]
    y = pos[:, 1]
    z = pos[:, 2]
    b = batch.astype(jnp.int32)

    mesh = plsc.VectorSubcoreMesh(core_axis_name="c", subcore_axis_name="s")
    run = functools.partial(
        pl.kernel,
        out_type=[
            jax.ShapeDtypeStruct((N * K,), jnp.int32),
            jax.ShapeDtypeStruct((N * K,), jnp.int32),
            jax.ShapeDtypeStruct((N * K,), jnp.float32),
        ],
        mesh=mesh,
        compiler_params=pltpu.CompilerParams(needs_layout_passes=False),
        scratch_types=[
            pltpu.VMEM((N,), jnp.float32),
            pltpu.VMEM((N,), jnp.float32),
            pltpu.VMEM((N,), jnp.float32),
            pltpu.VMEM((N + 2 * L,), jnp.int32),
            pltpu.VMEM((N,), jnp.float32),
            pltpu.VMEM((NB,), jnp.int32),
            pltpu.VMEM((NB,), jnp.int32),
            pltpu.VMEM((RPW * K,), jnp.int32),
            pltpu.VMEM((RPW * K,), jnp.int32),
            pltpu.VMEM((RPW * K,), jnp.float32),
        ],
    )(_tec_body)
    src, tgt, w = run(x, y, z, b)
    edge_index = jnp.stack([src, tgt])
    return edge_index, w


# speedup vs baseline: 128.5143x; 1.0158x over previous
import functools

import jax
import jax.numpy as jnp
from jax import lax
from jax.experimental import pallas as pl
from jax.experimental.pallas import tpu as pltpu
from jax.experimental.pallas import tpu_sc as plsc

N = 4096
K = 32
R2 = 25.0
NB = 32
L = 16
NC, NS = 2, 16
NW = NC * NS
RPW = N // NW
NCHUNK = N // L
INF = float("inf")


def _lexless(ka, va, kb, vb):
    return (ka < kb) | ((ka == kb) & (va < vb))


def _merge16to32(Ck, Cv, Dk, Dv):
    rDk = jnp.flip(Dk, 0)
    rDv = jnp.flip(Dv, 0)
    lt = _lexless(Ck, Cv, rDk, rDv)
    P0k = jnp.where(lt, Ck, rDk)
    P0v = jnp.where(lt, Cv, rDv)
    P1k = jnp.where(lt, rDk, Ck)
    P1v = jnp.where(lt, rDv, Cv)
    S0k, S0v = plsc.sort_key_val(P0k, P0v)
    S1k, S1v = plsc.sort_key_val(P1k, P1v)
    return S0k, S0v, S1k, S1v


def _merge32keep32(T0k, T0v, T1k, T1v, S0k, S0v, S1k, S1v):
    rS0k = jnp.flip(S0k, 0)
    rS0v = jnp.flip(S0v, 0)
    rS1k = jnp.flip(S1k, 0)
    rS1v = jnp.flip(S1v, 0)
    ltA = _lexless(T0k, T0v, rS1k, rS1v)
    L0k = jnp.where(ltA, T0k, rS1k)
    L0v = jnp.where(ltA, T0v, rS1v)
    ltB = _lexless(T1k, T1v, rS0k, rS0v)
    L1k = jnp.where(ltB, T1k, rS0k)
    L1v = jnp.where(ltB, T1v, rS0v)
    lt2 = _lexless(L0k, L0v, L1k, L1v)
    P0k = jnp.where(lt2, L0k, L1k)
    P0v = jnp.where(lt2, L0v, L1v)
    P1k = jnp.where(lt2, L1k, L0k)
    P1v = jnp.where(lt2, L1v, L0v)
    T0k, T0v = plsc.sort_key_val(P0k, P0v)
    T1k, T1v = plsc.sort_key_val(P1k, P1v)
    return T0k, T0v, T1k, T1v


def _sqrt16(x):
    i = plsc.bitcast(x, jnp.int32)
    i = jnp.int32(0x5F3759DF) - (i >> 1)
    y = plsc.bitcast(i, jnp.float32)
    half_x = jnp.float32(0.5) * x
    for _ in range(3):
        y = y * (jnp.float32(1.5) - half_x * y * y)
    return x * y


def _tec_body(x_hbm, y_hbm, z_hbm, b_hbm, src_hbm, tgt_hbm, w_hbm,
              xv, yv, zv, bv, sqv, lo_t, hi_t, src_v, tgt_v, w_v):
    wid = lax.axis_index("s") * NC + lax.axis_index("c")
    r0 = wid * RPW
    iota = lax.iota(jnp.int32, L)

    pltpu.sync_copy(x_hbm, xv)
    pltpu.sync_copy(y_hbm, yv)
    pltpu.sync_copy(z_hbm, zv)
    bv[pl.ds(0, L)] = jnp.full((L,), -1, jnp.int32)
    bv[pl.ds(L + N, L)] = jnp.full((L,), NB, jnp.int32)
    pltpu.sync_copy(b_hbm, bv.at[pl.ds(L, N)])

    def sq_body(i, carry):
        off = i * L
        x = xv[pl.ds(off, L)]
        y = yv[pl.ds(off, L)]
        z = zv[pl.ds(off, L)]
        sqv[pl.ds(off, L)] = x * x + y * y + z * z
        return carry

    lax.fori_loop(0, NCHUNK, sq_body, 0)

    def bnd_body(c, carry):
        off = c * L
        cur = bv[pl.ds(L + off, L)]
        prv = bv[pl.ds(L + off - 1, L)]
        nxt = bv[pl.ds(L + off + 1, L)]
        gidx = iota + off
        plsc.store_scatter(lo_t, [cur], gidx, mask=cur != prv)
        plsc.store_scatter(hi_t, [cur], gidx + 1, mask=cur != nxt)
        return carry

    lax.fori_loop(0, NCHUNK, bnd_body, 0)

    z16 = jnp.zeros((L,), jnp.int32)
    inf16 = jnp.full((L,), INF, jnp.float32)

    def rowpair_body(rp):
        gs = [r0 + 4 * rp + j for j in range(4)]

        def row_ctx(gi):
            gisp = jnp.broadcast_to(gi, (L,))
            xi = plsc.load_gather(xv, [gisp])
            yi = plsc.load_gather(yv, [gisp])
            zi = plsc.load_gather(zv, [gisp])
            sqi = plsc.load_gather(sqv, [gisp])
            bsp = plsc.load_gather(bv, [gisp + L])
            lo = jnp.max(plsc.load_gather(lo_t, [bsp]))
            hi = jnp.max(plsc.load_gather(hi_t, [bsp]))
            return gisp, xi, yi, zi, sqi, bsp, lo, hi

        ctxs = [row_ctx(g) for g in gs]
        lo = ctxs[0][6]
        hi = ctxs[0][7]
        for ctx in ctxs[1:]:
            lo = jnp.minimum(lo, ctx[6])
            hi = jnp.maximum(hi, ctx[7])
        c0 = lo // L
        c1 = (hi + (L - 1)) // L

        def chunk16(ctx, off, vidx, bx, by, bz, bb, bsq, cvalid):
            gisp, xi, yi, zi, sqi, bsp = ctx[:6]
            dot = xi * bx + yi * by + zi * bz
            d2 = jnp.maximum((sqi + bsq) - 2.0 * dot, 0.0)
            m = (bb == bsp) & (vidx != gisp) & (d2 <= R2) & cvalid
            key = jnp.where(m, d2, INF)
            return plsc.sort_key_val(key, vidx)

        def pair_body(p, T):
            ca = c0 + 2 * p
            cb = jnp.minimum(ca + 1, NCHUNK - 1)
            bvalid = ca + 1 < c1
            offa = ca * L
            offb = cb * L
            la = (xv[pl.ds(offa, L)], yv[pl.ds(offa, L)], zv[pl.ds(offa, L)],
                  bv[pl.ds(L + offa, L)], sqv[pl.ds(offa, L)])
            lb = (xv[pl.ds(offb, L)], yv[pl.ds(offb, L)], zv[pl.ds(offb, L)],
                  bv[pl.ds(L + offb, L)], sqv[pl.ds(offb, L)])
            via = iota + offa
            vib = iota + offb
            Tn = []
            for j, ctx in enumerate(ctxs):
                Ck, Cv = chunk16(ctx, offa, via, *la, True)
                Dk, Dv = chunk16(ctx, offb, vib, *lb, bvalid)
                S = _merge16to32(Ck, Cv, Dk, Dv)
                Tn.extend(_merge32keep32(*T[4 * j:4 * j + 4], *S))
            return tuple(Tn)

        npairs = (c1 - c0 + 1) // 2
        init = (inf16, z16, inf16, z16)
        T = lax.fori_loop(0, npairs, pair_body, init * 4)

        for j, ctx in enumerate(ctxs):
            gisp_r = ctx[0]
            rr = 4 * rp + j
            T0k, T0v, T1k, T1v = T[4 * j:4 * j + 4]
            base = rr * K
            for half, (tk, tv) in enumerate(((T0k, T0v), (T1k, T1v))):
                keep = tk <= R2
                good = keep & (tk > 0.0)
                safe = jnp.where(good, tk, jnp.float32(1.0))
                wgt = jnp.where(good, _sqrt16(safe), jnp.float32(0.0))
                off = base + half * L
                src_v[pl.ds(off, L)] = jnp.where(keep, tv, -1)
                tgt_v[pl.ds(off, L)] = jnp.where(keep, gisp_r, -1)
                w_v[pl.ds(off, L)] = wgt

    plsc.parallel_loop(0, RPW // 4, 1, unroll=1)(rowpair_body)

    out0 = r0 * K
    pltpu.sync_copy(src_v, src_hbm.at[pl.ds(out0, RPW * K)])
    pltpu.sync_copy(tgt_v, tgt_hbm.at[pl.ds(out0, RPW * K)])
    pltpu.sync_copy(w_v, w_hbm.at[pl.ds(out0, RPW * K)])


@jax.jit
def kernel(pos, batch):
    x = pos[:, 0]
    y = pos[:, 1]
    z = pos[:, 2]
    b = batch.astype(jnp.int32)

    mesh = plsc.VectorSubcoreMesh(core_axis_name="c", subcore_axis_name="s")
    run = functools.partial(
        pl.kernel,
        out_type=[
            jax.ShapeDtypeStruct((N * K,), jnp.int32),
            jax.ShapeDtypeStruct((N * K,), jnp.int32),
            jax.ShapeDtypeStruct((N * K,), jnp.float32),
        ],
        mesh=mesh,
        compiler_params=pltpu.CompilerParams(needs_layout_passes=False),
        scratch_types=[
            pltpu.VMEM((N,), jnp.float32),
            pltpu.VMEM((N,), jnp.float32),
            pltpu.VMEM((N,), jnp.float32),
            pltpu.VMEM((N + 2 * L,), jnp.int32),
            pltpu.VMEM((N,), jnp.float32),
            pltpu.VMEM((NB,), jnp.int32),
            pltpu.VMEM((NB,), jnp.int32),
            pltpu.VMEM((RPW * K,), jnp.int32),
            pltpu.VMEM((RPW * K,), jnp.int32),
            pltpu.VMEM((RPW * K,), jnp.float32),
        ],
    )(_tec_body)
    src, tgt, w = run(x, y, z, b)
    edge_index = jnp.stack([src, tgt])
    return edge_index, w
